# baseline (speedup 1.0000x reference)
import functools

import jax
import jax.numpy as jnp
from jax import lax
from jax.experimental import pallas as pl
from jax.experimental.pallas import tpu as pltpu
from jax.experimental.pallas import tpu_sc as plsc

NC = 2
NS = 16
NW = NC * NS
L = 16

GR = 16


@functools.lru_cache(maxsize=None)
def _build_repack(V, D):
    VP = V // 4
    n_chunks = VP // GR
    per_w = n_chunks // NW
    n_used = NW * per_w
    tail_chunks = n_chunks - n_used
    assert VP % GR == 0 and per_w % 2 == 0 and tail_chunks < NW
    mesh = plsc.VectorSubcoreMesh(core_axis_name="c", subcore_axis_name="s")

    @functools.partial(
        pl.kernel,
        mesh=mesh,
        out_type=jax.ShapeDtypeStruct((VP, 4 * D), jnp.float32),
        scratch_types=[
            pltpu.VMEM((4 * GR, D), jnp.float32),
            pltpu.VMEM((4 * GR, D), jnp.float32),
            pltpu.VMEM((GR, 4 * D), jnp.float32),
            pltpu.VMEM((GR, 4 * D), jnp.float32),
            pltpu.SemaphoreType.DMA,
            pltpu.SemaphoreType.DMA,
            pltpu.SemaphoreType.DMA,
            pltpu.SemaphoreType.DMA,
        ],
    )
    def repack_kernel(table_hbm, packed_hbm,
                      stag0, stag1, wbuf0, wbuf1,
                      isem0, isem1, osem0, osem1):
        stags = (stag0, stag1)
        wbufs = (wbuf0, wbuf1)
        isems = (isem0, isem1)
        osems = (osem0, osem1)

        wid = lax.axis_index("s") * NC + lax.axis_index("c")
        base_p = wid * per_w * GR

        def fire_in(i, s):
            r0 = (base_p + i * GR) * 4
            pltpu.async_copy(
                table_hbm.at[pl.ds(r0, 4 * GR)], stags[s], isems[s]
            )

        def drain_in(s):
            pltpu.make_async_copy(
                table_hbm.at[pl.ds(0, 4 * GR)], stags[s], isems[s]
            ).wait()

        def compact(s):
            stag, wbuf = stags[s], wbufs[s]
            for p in range(GR):
                for k in range(4):
                    r = 4 * p + k
                    wbuf[p, pl.ds(k * D, L)] = stag[r, pl.ds(0, L)]
                    wbuf[p, pl.ds(k * D + L, L)] = stag[r, pl.ds(L, L)]

        def fire_out(i, s):
            pltpu.async_copy(
                wbufs[s], packed_hbm.at[pl.ds(base_p + i * GR, GR)], osems[s]
            )

        def drain_out(s):
            pltpu.make_async_copy(
                wbufs[s], packed_hbm.at[pl.ds(base_p, GR)], osems[s]
            ).wait()

        fire_in(0, 0)

        def body(i2, _):
            for k in range(2):
                i = 2 * i2 + k
                s = k
                drain_in(s)

                @pl.when(i + 1 < per_w)
                def _():
                    fire_in(i + 1, 1 - s)

                @pl.when(i >= 2)
                def _():
                    drain_out(s)

                compact(s)
                fire_out(i, s)
            return ()

        lax.fori_loop(0, per_w // 2, body, ())

        drain_out(0)
        drain_out(1)

        if tail_chunks:

            @pl.when(wid < tail_chunks)
            def _():
                p0 = (n_used + wid) * GR
                pltpu.sync_copy(table_hbm.at[pl.ds(p0 * 4, 4 * GR)], stags[0])
                compact(0)
                pltpu.sync_copy(wbufs[0], packed_hbm.at[pl.ds(p0, GR)])

    return repack_kernel


@functools.lru_cache(maxsize=None)
def _build_gather(B, F, V, D, NB):
    N = B * F
    CH = NB * F
    per_w_b = B // NW
    per_w = per_w_b * F
    n_ch = per_w_b // NB
    n_grp = CH // L
    assert CH % L == 0 and n_ch % 2 == 0
    mesh = plsc.VectorSubcoreMesh(core_axis_name="c", subcore_axis_name="s")

    @functools.partial(
        pl.kernel,
        mesh=mesh,
        out_type=jax.ShapeDtypeStruct((B, F, D), jnp.float32),
        scratch_types=[
            pltpu.VMEM((CH,), jnp.int32),
            pltpu.VMEM((CH,), jnp.int32),
            pltpu.VMEM((CH,), jnp.int32),
            pltpu.VMEM((CH,), jnp.int32),
            pltpu.VMEM((CH, 4 * D), jnp.float32),
            pltpu.VMEM((CH, 4 * D), jnp.float32),
            pltpu.VMEM((CH, D), jnp.float32),
            pltpu.VMEM((CH, D), jnp.float32),
            pltpu.SemaphoreType.DMA,
            pltpu.SemaphoreType.DMA,
            pltpu.SemaphoreType.DMA,
            pltpu.SemaphoreType.DMA,
        ],
    )
    def gather_kernel(idx_hbm, table_hbm, out_hbm,
                      idx_v0, idx_v1, hi_v0, hi_v1,
                      rows_v0, rows_v1, out_v0, out_v1,
                      gsem0, gsem1, wsem0, wsem1):
        idx_vs = (idx_v0, idx_v1)
        hi_vs = (hi_v0, hi_v1)
        rows_vs = (rows_v0, rows_v1)
        out_vs = (out_v0, out_v1)
        gsems = (gsem0, gsem1)
        wsems = (wsem0, wsem1)

        wid = lax.axis_index("s") * NC + lax.axis_index("c")
        base_row = wid * per_w
        base_b = wid * per_w_b

        def fire(i, s):
            idx_v, hi_v, rows_v = idx_vs[s], hi_vs[s], rows_vs[s]
            off = base_row + i * CH
            pltpu.sync_copy(idx_hbm.at[pl.ds(off, CH)], idx_v)

            def grp(g, _):
                vec = idx_v[pl.ds(g * L, L)]
                hi_v[pl.ds(g * L, L)] = lax.shift_right_logical(vec, 2)
                return ()

            lax.fori_loop(0, n_grp, grp, ())
            pltpu.async_copy(table_hbm.at[hi_v], rows_v, gsems[s])

        def drain_gather(s):
            pltpu.make_async_copy(
                table_hbm.at[pl.ds(0, CH)], rows_vs[s], gsems[s]
            ).wait()

        def extract(s):
            idx_v, rows_v, out_v = idx_vs[s], rows_vs[s], out_vs[s]

            def grp(g, _):
                vec = idx_v[pl.ds(g * L, L)]
                for j in range(L):
                    r = g * L + j
                    q = lax.shift_left(vec[j] & 3, 5)
                    out_v[r, pl.ds(0, L)] = rows_v[r, pl.ds(q, L)]
                    out_v[r, pl.ds(L, L)] = rows_v[r, pl.ds(q + L, L)]
                return ()

            lax.fori_loop(0, n_grp, grp, ())

        def writeback(i, s):
            bb = base_b + i * NB
            for k in range(NB):
                pltpu.async_copy(
                    out_vs[s].at[pl.ds(k * F, F)],
                    out_hbm.at[bb + k],
                    wsems[s],
                )

        def drain_write(s):
            for _ in range(NB):
                pltpu.make_async_copy(
                    out_vs[s].at[pl.ds(0, F)], out_hbm.at[base_b], wsems[s]
                ).wait()

        fire(0, 0)

        def body(i2, _):
            for k in range(2):
                i = 2 * i2 + k
                s = k
                drain_gather(s)

                @pl.when(i + 1 < n_ch)
                def _():
                    fire(i + 1, 1 - s)

                @pl.when(i >= 2)
                def _():
                    drain_write(s)

                extract(s)
                writeback(i, s)
            return ()

        lax.fori_loop(0, n_ch // 2, body, ())

        drain_write(0)
        drain_write(1)

    return gather_kernel


def kernel(feature, weight):
    B, F = feature.shape
    V, D = weight.shape
    idx = feature.reshape(B * F).astype(jnp.int32)
    packed = _build_repack(V, D)(weight)
    return _build_gather(B, F, V, D, 8)(idx, packed)


# speedup vs baseline: 1.7193x; 1.7193x over previous
import functools

import jax
import jax.numpy as jnp
from jax import lax
from jax.experimental import pallas as pl
from jax.experimental.pallas import tpu as pltpu
from jax.experimental.pallas import tpu_sc as plsc

NC = 2
NS = 16
NW = NC * NS
L = 16


@functools.lru_cache(maxsize=None)
def _build(B, F, V, D, NB):
    N = B * F
    CH = NB * F
    per_w_b = B // NW
    per_w = per_w_b * F
    n_ch = per_w_b // NB
    assert n_ch % 2 == 0
    mesh = plsc.VectorSubcoreMesh(core_axis_name="c", subcore_axis_name="s")

    @functools.partial(
        pl.kernel,
        mesh=mesh,
        out_type=jax.ShapeDtypeStruct((B, 32, 128), jnp.float32),
        compiler_params=pltpu.CompilerParams(use_tc_tiling_on_sc=False),
        scratch_types=[
            pltpu.VMEM((CH,), jnp.int32),
            pltpu.VMEM((CH,), jnp.int32),
            pltpu.VMEM((CH, D), jnp.float32),
            pltpu.VMEM((CH, D), jnp.float32),
            pltpu.SemaphoreType.DMA,
            pltpu.SemaphoreType.DMA,
            pltpu.SemaphoreType.DMA,
            pltpu.SemaphoreType.DMA,
        ],
    )
    def gather_kernel(idx_hbm, table_hbm, out_hbm,
                      idx_v0, idx_v1, rows_v0, rows_v1,
                      gsem0, gsem1, wsem0, wsem1):
        idx_vs = (idx_v0, idx_v1)
        rows_vs = (rows_v0, rows_v1)
        gsems = (gsem0, gsem1)
        wsems = (wsem0, wsem1)

        wid = lax.axis_index("s") * NC + lax.axis_index("c")
        base_row = wid * per_w
        base_b = wid * per_w_b

        def fire(i, s):
            off = base_row + i * CH
            pltpu.sync_copy(idx_hbm.at[pl.ds(off, CH)], idx_vs[s])
            pltpu.async_copy(table_hbm.at[idx_vs[s]], rows_vs[s], gsems[s])

        def drain_gather(s):
            pltpu.make_async_copy(
                table_hbm.at[pl.ds(0, CH)], rows_vs[s], gsems[s]
            ).wait()

        def writeback(i, s):
            bb = base_b + i * NB
            for k in range(NB):
                pltpu.async_copy(
                    rows_vs[s].at[pl.ds(k * F, F)],
                    out_hbm.at[bb + k, pl.ds(0, F), pl.ds(0, D)],
                    wsems[s],
                )

        def drain_write(s):
            for _ in range(NB):
                pltpu.make_async_copy(
                    rows_vs[s].at[pl.ds(0, F)],
                    out_hbm.at[base_b, pl.ds(0, F), pl.ds(0, D)],
                    wsems[s],
                ).wait()

        fire(0, 0)

        def body(i2, _):
            for k in range(2):
                i = 2 * i2 + k
                s = k
                drain_gather(s)
                writeback(i, s)

                @pl.when(i + 1 < n_ch)
                def _():
                    @pl.when(i >= 1)
                    def _():
                        drain_write(1 - s)

                    fire(i + 1, 1 - s)
            return ()

        lax.fori_loop(0, n_ch // 2, body, ())

        drain_write(0)
        drain_write(1)

    return gather_kernel


def kernel(feature, weight):
    B, F = feature.shape
    V, D = weight.shape
    idx = feature.reshape(B * F).astype(jnp.int32)
    out = _build(B, F, V, D, 8)(idx, weight)
    return out[:, :F, :D]
